# baseline (speedup 1.0000x reference)
import jax
import jax.numpy as jnp
import numpy as np
from jax.experimental import pallas as pl
from jax.experimental.pallas import tpu as pltpu

_N = 512
_HID = 32
_HARM = 5
_NODE_IN = 8
_NBLK = 5
_CI = 8
_CW = _CI * _N
_NC = _N // _CI

_T0 = (((0,), (0,)), ((), ()))


def _gnn_body(xcols, kmul, selo, sels, selc, kan_w, kan_b,
              ew1t, eb1t, ew2t, eb2t, ew3t, eb3t,
              am, e8, i512, i32,
              ge1t, geb1t, ge2t, geb2t, ge3t, geb3t,
              wn1, wn2, wn3, bn1, bn2, bn3,
              wd1, wd2, wd3, bd1, bd2, bd3,
              out, edge_s, node_s, agg_s):
    f32 = jnp.float32

    def mm(a, b):
        return jnp.dot(a, b, preferred_element_type=f32)

    def mmx(a, b):
        return jnp.dot(a, b, preferred_element_type=f32,
                       precision=jax.lax.Precision.HIGHEST)

    def dgT(a, b):
        return jax.lax.dot_general(a, b, _T0,
                                   preferred_element_type=f32)

    y = xcols[...] * kmul[...]
    basis = selo[...] + sels[...] * jnp.sin(y) + selc[...] * jnp.cos(y)
    kb = jnp.sum(kan_b[...], axis=0, keepdims=True)
    node_s[...] = mm(basis, kan_w[...]) + kb

    ones41 = jnp.ones((4, 1), f32)
    h = jnp.maximum(mmx(ew1t[...], ones41) + eb1t[...], 0.0)
    h = h.astype(jnp.bfloat16).astype(f32)
    h = jnp.maximum(mm(ew2t[...], h) + eb2t[...], 0.0)
    h = h.astype(jnp.bfloat16).astype(f32)
    e0t = mm(ew3t[...], h) + eb3t[...]

    for l in range(_NBLK):
        node_n = node_s[...]
        node_t = dgT(node_n, i512[...])
        agg_s[...] = jnp.zeros((_HID, _N), f32)

        def chunk(c, carry, l=l):
            cols = pl.ds(c * _CW, _CW)
            snd = node_s[pl.ds(c * _CI, _CI), :]
            sterm = dgT(snd, e8[...])
            rterm = jnp.concatenate([node_t] * _CI, axis=1)
            if l == 0:
                old = jnp.broadcast_to(e0t, (_HID, _CW))
            else:
                old = edge_s[:, cols]
            xin = jnp.concatenate([old, sterm, rterm], axis=0)
            h1 = jnp.maximum(mm(ge1t[l], xin) + geb1t[l], 0.0)
            h2 = jnp.maximum(mm(ge2t[l], h1) + geb2t[l], 0.0)
            d = mm(ge3t[l], h2) + geb3t[l]
            new = old + d
            if l < _NBLK - 1:
                edge_s[:, cols] = new
            mn = am[pl.ds(c, 1), :] * new
            acc = agg_s[...]
            for i in range(_CI):
                acc = acc + mn[:, i * _N:(i + 1) * _N]
            agg_s[...] = acc
            return carry

        jax.lax.fori_loop(0, _NC, chunk, 0)

        agg_n = dgT(agg_s[...], i32[...])
        ni = jnp.concatenate([node_n, agg_n], axis=1)
        h1 = jnp.maximum(mm(ni, wn1[l]) + bn1[l][None, :], 0.0)
        h2 = jnp.maximum(mm(h1, wn2[l]) + bn2[l][None, :], 0.0)
        node_s[...] = node_n + mm(h2, wn3[l]) + bn3[l][None, :]

    node_n = node_s[...]
    h1 = jnp.maximum(mm(node_n, wd1[...]) + bd1[...], 0.0)
    h2 = jnp.maximum(mm(h1, wd2[...]) + bd2[...], 0.0)
    out[...] = mm(h2, wd3[...]) + bd3[...]


_NBAS = 2 * _HARM + 1
_COL_I = np.repeat(np.arange(_NODE_IN), _NBAS)
_T = np.tile(np.arange(_NBAS), _NODE_IN)
_KMUL = np.tile(
    np.array([0, 1, 1, 2, 2, 3, 3, 4, 4, 5, 5], np.float32), _NODE_IN)
_SELO = (_T == 0).astype(np.float32)
_SELS = (_T % 2 == 1).astype(np.float32)
_SELC = ((_T > 0) & (_T % 2 == 0)).astype(np.float32)

_E8 = (np.arange(_CW)[None, :] // _N == np.arange(_CI)[:, None]
       ).astype(np.float32)
_I512 = np.eye(_N, dtype=np.float32)
_I32 = np.eye(_HID, dtype=np.float32)


def kernel(x, adj, kan_W, kan_b, enc_W1, enc_b1, enc_W2, enc_b2, enc_W3,
           enc_b3, gn_eW1, gn_eb1, gn_eW2, gn_eb2, gn_eW3, gn_eb3,
           gn_nW1, gn_nb1, gn_nW2, gn_nb2, gn_nW3, gn_nb3,
           dec_W1, dec_b1, dec_W2, dec_b2, dec_W3, dec_b3):
    node_x = x[0, -1]
    a = adj[0]

    am = a.astype(jnp.float32).reshape(_NC, _CW)
    xcols = node_x[:, _COL_I]
    kan_w_flat = kan_W.reshape(_NODE_IN * _NBAS, _HID)

    tT = lambda w: jnp.transpose(w, (0, 2, 1))
    col = lambda b: b[:, :, None] if b.ndim == 2 else b[:, None]

    out = pl.pallas_call(
        _gnn_body,
        out_shape=jax.ShapeDtypeStruct((_N, dec_b3.shape[0]), jnp.float32),
        scratch_shapes=[
            pltpu.VMEM((_HID, _N * _N), jnp.float32),
            pltpu.VMEM((_N, _HID), jnp.float32),
            pltpu.VMEM((_HID, _N), jnp.float32),
        ],
        compiler_params=pltpu.CompilerParams(
            vmem_limit_bytes=100 * 1024 * 1024),
    )(
        xcols, _KMUL[None, :], _SELO[None, :], _SELS[None, :],
        _SELC[None, :], kan_w_flat, kan_b,
        enc_W1.T, col(enc_b1), enc_W2.T, col(enc_b2), enc_W3.T,
        col(enc_b3),
        am, _E8, _I512, _I32,
        tT(gn_eW1), col(gn_eb1), tT(gn_eW2), col(gn_eb2), tT(gn_eW3),
        col(gn_eb3),
        gn_nW1, gn_nW2, gn_nW3, gn_nb1, gn_nb2, gn_nb3,
        dec_W1, dec_W2, dec_W3,
        dec_b1[None, :], dec_b2[None, :], dec_b3[None, :],
    )
    return out[None]
